# baseline (speedup 1.0000x reference)
import functools

import jax
import jax.numpy as jnp
from jax import lax
from jax.experimental import pallas as pl
from jax.experimental.pallas import tpu as pltpu
from jax.experimental.pallas import tpu_sc as plsc

_NC = 2
_NS = 16
_NW = _NC * _NS
_CHUNK = 128


def _sc_gather(table, idx2d):
    n_rows, chunk = idx2d.shape
    batch = n_rows * chunk
    d = table.shape[1]
    rows_per_w = n_rows // _NW
    b_per_w = batch // _NW

    mesh = plsc.VectorSubcoreMesh(core_axis_name="c", subcore_axis_name="s")

    @functools.partial(
        pl.kernel,
        mesh=mesh,
        compiler_params=pltpu.CompilerParams(use_tc_tiling_on_sc=False),
        out_type=jax.ShapeDtypeStruct((batch, d), jnp.float32),
        scratch_types=[
            pltpu.VMEM((rows_per_w, chunk), jnp.int32),
            pltpu.VMEM((b_per_w, d), jnp.float32),
            pltpu.SemaphoreType.DMA,
        ],
    )
    def gather(table_hbm, idx_hbm, out_hbm, idx_v, rows_v, sem):
        wid = lax.axis_index("s") * _NC + lax.axis_index("c")
        row_base = wid * rows_per_w
        pltpu.sync_copy(idx_hbm.at[pl.ds(row_base, rows_per_w)], idx_v)
        copies = []
        for i in range(rows_per_w):
            copies.append(
                pltpu.async_copy(
                    table_hbm.at[idx_v.at[i]],
                    rows_v.at[pl.ds(i * chunk, chunk)],
                    sem,
                )
            )
        for cp in copies:
            cp.wait()
        pltpu.sync_copy(rows_v, out_hbm.at[pl.ds(wid * b_per_w, b_per_w)])

    return gather(table, idx2d)


def _tc_project(hidden, wt, b2d):
    batch, d = hidden.shape
    o = wt.shape[1]
    blk = 2048
    grid = batch // blk

    def body(h_ref, w_ref, b_ref, o_ref):
        o_ref[...] = (
            jnp.dot(h_ref[...], w_ref[...], preferred_element_type=jnp.float32)
            + b_ref[...]
        )

    return pl.pallas_call(
        body,
        grid=(grid,),
        in_specs=[
            pl.BlockSpec((blk, d), lambda i: (i, 0)),
            pl.BlockSpec((d, o), lambda i: (0, 0)),
            pl.BlockSpec((1, o), lambda i: (0, 0)),
        ],
        out_specs=pl.BlockSpec((blk, o), lambda i: (i, 0)),
        out_shape=jax.ShapeDtypeStruct((batch, o), jnp.float32),
    )(hidden, wt, b2d)


def kernel(x_id, table, W, b):
    idx2d = x_id.astype(jnp.int32).reshape(-1, _CHUNK)
    hidden = _sc_gather(table, idx2d)
    return _tc_project(hidden, W.T, b.reshape(1, -1))


# speedup vs baseline: 2.6502x; 2.6502x over previous
import functools

import jax
import jax.numpy as jnp
from jax import lax
from jax.experimental import pallas as pl
from jax.experimental.pallas import tpu as pltpu
from jax.experimental.pallas import tpu_sc as plsc

_NC = 2
_NS = 16
_NW = _NC * _NS
_CHUNK = 128
_BLK = 4096


def _tc_project_table(tableT, W, b):
    d, v = tableT.shape
    o = W.shape[0]
    grid = (v + _BLK - 1) // _BLK

    def body(t_ref, w_ref, b_ref, *o_refs):
        res = lax.dot_general(
            w_ref[...], t_ref[...], (((1,), (0,)), ((), ())),
            preferred_element_type=jnp.float32,
        )
        for j in range(o):
            o_refs[j][...] = res[j, :] + b_ref[j]

    return pl.pallas_call(
        body,
        grid=(grid,),
        in_specs=[
            pl.BlockSpec((d, _BLK), lambda i: (0, i)),
            pl.BlockSpec((o, d), lambda i: (0, 0)),
            pl.BlockSpec(memory_space=pltpu.SMEM),
        ],
        out_specs=tuple(pl.BlockSpec((_BLK,), lambda i: (i,)) for _ in range(o)),
        out_shape=tuple(
            jax.ShapeDtypeStruct((v,), jnp.float32) for _ in range(o)
        ),
    )(tableT, W, b)


def _sc_gather_cols(cols, idx2d):
    o = len(cols)
    n_rows, chunk = idx2d.shape
    batch = n_rows * chunk
    rows_per_w = n_rows // _NW
    b_per_w = batch // _NW

    mesh = plsc.VectorSubcoreMesh(core_axis_name="c", subcore_axis_name="s")

    @functools.partial(
        pl.kernel,
        mesh=mesh,
        compiler_params=pltpu.CompilerParams(use_tc_tiling_on_sc=False),
        out_type=jax.ShapeDtypeStruct((o, batch), jnp.float32),
        scratch_types=[
            pltpu.VMEM((rows_per_w, chunk), jnp.int32),
            pltpu.VMEM((o, b_per_w), jnp.float32),
            pltpu.SemaphoreType.DMA,
        ],
    )
    def gather(*refs):
        col_hbms = refs[:o]
        idx_hbm = refs[o]
        out_hbm = refs[o + 1]
        idx_v, vals_v, sem = refs[o + 2], refs[o + 3], refs[o + 4]
        wid = lax.axis_index("s") * _NC + lax.axis_index("c")
        row_base = wid * rows_per_w
        pltpu.sync_copy(idx_hbm.at[pl.ds(row_base, rows_per_w)], idx_v)
        copies = []
        for j in range(o):
            for i in range(rows_per_w):
                copies.append(
                    pltpu.async_copy(
                        col_hbms[j].at[idx_v.at[i]],
                        vals_v.at[j, pl.ds(i * chunk, chunk)],
                        sem,
                    )
                )
        for cp in copies:
            cp.wait()
        pltpu.sync_copy(
            vals_v, out_hbm.at[:, pl.ds(wid * b_per_w, b_per_w)]
        )

    return gather(*cols, idx2d)


def kernel(x_id, table, W, b):
    tableT = table.T
    cols = _tc_project_table(tableT, W, b)
    idx2d = x_id.astype(jnp.int32).reshape(-1, _CHUNK)
    pout = _sc_gather_cols(cols, idx2d)
    return pout.T


# speedup vs baseline: 4.8397x; 1.8262x over previous
import functools

import jax
import jax.numpy as jnp
from jax import lax
from jax.experimental import pallas as pl
from jax.experimental.pallas import tpu as pltpu
from jax.experimental.pallas import tpu_sc as plsc

_NC = 2
_NS = 16
_NW = _NC * _NS
_CHUNK = 128
_BLK = 16384


def _tc_project_table(tableT, W, b):
    d, v = tableT.shape
    o = W.shape[0]
    grid = (v + _BLK - 1) // _BLK

    def body(t_ref, w_ref, b_ref, *o_refs):
        res = lax.dot_general(
            w_ref[...], t_ref[...], (((1,), (0,)), ((), ())),
            preferred_element_type=jnp.float32,
        )
        for j in range(o):
            o_refs[j][...] = res[j, :] + b_ref[j]

    return pl.pallas_call(
        body,
        grid=(grid,),
        in_specs=[
            pl.BlockSpec((d, _BLK), lambda i: (0, i)),
            pl.BlockSpec((o, d), lambda i: (0, 0)),
            pl.BlockSpec(memory_space=pltpu.SMEM),
        ],
        out_specs=tuple(pl.BlockSpec((_BLK,), lambda i: (i,)) for _ in range(o)),
        out_shape=tuple(
            jax.ShapeDtypeStruct((v,), jnp.float32) for _ in range(o)
        ),
    )(tableT, W, b)


def _sc_gather_cols(cols, idx2d):
    o = len(cols)
    n_rows, chunk = idx2d.shape
    batch = n_rows * chunk
    rows_per_w = n_rows // _NW
    b_per_w = batch // _NW

    mesh = plsc.VectorSubcoreMesh(core_axis_name="c", subcore_axis_name="s")

    @functools.partial(
        pl.kernel,
        mesh=mesh,
        compiler_params=pltpu.CompilerParams(use_tc_tiling_on_sc=False),
        out_type=jax.ShapeDtypeStruct((o, batch), jnp.float32),
        scratch_types=[
            pltpu.VMEM((rows_per_w, chunk), jnp.int32),
            pltpu.VMEM((o, b_per_w), jnp.float32),
            pltpu.SemaphoreType.DMA,
        ],
    )
    def gather(*refs):
        col_hbms = refs[:o]
        idx_hbm = refs[o]
        out_hbm = refs[o + 1]
        idx_v, vals_v, sem = refs[o + 2], refs[o + 3], refs[o + 4]
        wid = lax.axis_index("s") * _NC + lax.axis_index("c")
        row_base = wid * rows_per_w
        pltpu.sync_copy(idx_hbm.at[pl.ds(row_base, rows_per_w)], idx_v)
        copies = []
        for j in range(o):
            for i in range(rows_per_w):
                copies.append(
                    pltpu.async_copy(
                        col_hbms[j].at[idx_v.at[i]],
                        vals_v.at[j, pl.ds(i * chunk, chunk)],
                        sem,
                    )
                )
        for cp in copies:
            cp.wait()
        pltpu.sync_copy(
            vals_v, out_hbm.at[:, pl.ds(wid * b_per_w, b_per_w)]
        )

    return gather(*cols, idx2d)


def kernel(x_id, table, W, b):
    tableT = table.T
    cols = _tc_project_table(tableT, W, b)
    idx2d = x_id.astype(jnp.int32).reshape(-1, _CHUNK)
    pout = _sc_gather_cols(cols, idx2d)
    return pout.T


# speedup vs baseline: 5.4987x; 1.1362x over previous
import functools

import jax
import jax.numpy as jnp
from jax import lax
from jax.experimental import pallas as pl
from jax.experimental.pallas import tpu as pltpu
from jax.experimental.pallas import tpu_sc as plsc

_NC = 2
_NS = 16
_NW = _NC * _NS
_CHUNK = 128
_BLK = 32768


def _tc_project_table(tableT, W, b):
    d, v = tableT.shape
    o = W.shape[0]
    grid = (v + _BLK - 1) // _BLK

    def body(t_ref, w_ref, b_ref, *o_refs):
        res = lax.dot_general(
            w_ref[...], t_ref[...], (((1,), (0,)), ((), ())),
            preferred_element_type=jnp.float32,
        )
        for j in range(o):
            o_refs[j][...] = res[j, :] + b_ref[j]

    return pl.pallas_call(
        body,
        grid=(grid,),
        in_specs=[
            pl.BlockSpec((d, _BLK), lambda i: (0, i)),
            pl.BlockSpec((o, d), lambda i: (0, 0)),
            pl.BlockSpec(memory_space=pltpu.SMEM),
        ],
        out_specs=tuple(pl.BlockSpec((_BLK,), lambda i: (i,)) for _ in range(o)),
        out_shape=tuple(
            jax.ShapeDtypeStruct((v,), jnp.float32) for _ in range(o)
        ),
    )(tableT, W, b)


def _sc_gather_cols(cols, idx2d):
    o = len(cols)
    n_rows, chunk = idx2d.shape
    batch = n_rows * chunk
    rows_per_w = n_rows // _NW
    b_per_w = batch // _NW

    mesh = plsc.VectorSubcoreMesh(core_axis_name="c", subcore_axis_name="s")

    @functools.partial(
        pl.kernel,
        mesh=mesh,
        compiler_params=pltpu.CompilerParams(use_tc_tiling_on_sc=False),
        out_type=jax.ShapeDtypeStruct((o, batch), jnp.float32),
        scratch_types=[
            pltpu.VMEM((rows_per_w, chunk), jnp.int32),
            pltpu.VMEM((o, b_per_w), jnp.float32),
            pltpu.SemaphoreType.DMA,
        ],
    )
    def gather(*refs):
        col_hbms = refs[:o]
        idx_hbm = refs[o]
        out_hbm = refs[o + 1]
        idx_v, vals_v, sem = refs[o + 2], refs[o + 3], refs[o + 4]
        wid = lax.axis_index("s") * _NC + lax.axis_index("c")
        row_base = wid * rows_per_w
        pltpu.sync_copy(idx_hbm.at[pl.ds(row_base, rows_per_w)], idx_v)
        copies = []
        for j in range(o):
            for i in range(rows_per_w):
                copies.append(
                    pltpu.async_copy(
                        col_hbms[j].at[idx_v.at[i]],
                        vals_v.at[j, pl.ds(i * chunk, chunk)],
                        sem,
                    )
                )
        for cp in copies:
            cp.wait()
        pltpu.sync_copy(
            vals_v, out_hbm.at[:, pl.ds(wid * b_per_w, b_per_w)]
        )

    return gather(*cols, idx2d)


def kernel(x_id, table, W, b):
    tableT = table.T
    cols = _tc_project_table(tableT, W, b)
    idx2d = x_id.astype(jnp.int32).reshape(-1, _CHUNK)
    pout = _sc_gather_cols(cols, idx2d)
    return pout.T
